# baseline (speedup 1.0000x reference)
import jax
import jax.numpy as jnp
from jax import lax
from jax.experimental import pallas as pl
from jax.experimental.pallas import tpu as pltpu
from jax.experimental.pallas import tpu_sc as plsc

EMB = 64
BATCH = 16384
NC = 2
NS = 16
NW = NC * NS
PER_W = BATCH // NW
CHUNK = 128
NCHUNK = PER_W // CHUNK
GROUPS = CHUNK // 16


def _body(s_hbm, o_hbm, r_hbm, t_hbm, e_hbm, re_hbm, te_hbm, out_hbm,
          s_idx, o_idx, r_idx, t_idx, acc0, acc1, ob0, ob1,
          res, sem_a0, sem_a1, sem_o0, sem_o1):
    wid = lax.axis_index("s") * NC + lax.axis_index("c")

    pltpu.sync_copy(s_hbm.at[wid], s_idx)
    pltpu.sync_copy(o_hbm.at[wid], o_idx)
    pltpu.sync_copy(r_hbm.at[wid], r_idx)
    pltpu.sync_copy(t_hbm.at[wid], t_idx)

    accs = (acc0, acc1)
    obs = (ob0, ob1)
    sems_a = (sem_a0, sem_a1)
    sems_o = (sem_o0, sem_o1)

    iota = lax.iota(jnp.int32, 16)

    def fire_ro(ch, b):
        cr = pltpu.async_copy(re_hbm.at[r_idx.at[ch]], accs[b], sems_a[b])
        co = pltpu.async_copy(e_hbm.at[o_idx.at[ch]], obs[b], sems_o[b])
        return cr, co

    def fire_t(ch, b):
        return pltpu.async_copy(te_hbm.at[t_idx.at[ch]], accs[b], sems_a[b],
                                add=True)

    def fire_s(ch, b):
        return pltpu.async_copy(e_hbm.at[s_idx.at[ch]], accs[b], sems_a[b],
                                add=True)

    def group(ch, b, g):
        rid = iota + (g * 16)

        def col_body(j, carry):
            acc, col = carry
            va = plsc.load_gather(accs[b], [rid, col])
            vo = plsc.load_gather(obs[b], [rid, col])
            return acc + jnp.abs(va - vo), (col + 1) & 63

        (acc, _) = plsc.parallel_loop(
            0, EMB, carry=(jnp.zeros((16,), jnp.float32), iota),
            unroll=8)(col_body)
        res[pl.ds(ch * CHUNK + g * 16, 16)] = acc

    cr, co = fire_ro(0, 0)
    cr.wait()
    fire_t(0, 0).wait()
    fire_s(0, 0).wait()
    co.wait()

    b = 0
    for ch in range(NCHUNK):
        nxt = ch + 1
        nb = 1 - b
        if nxt < NCHUNK:
            crn, con = fire_ro(nxt, nb)
        group(ch, b, 0)
        group(ch, b, 1)
        group(ch, b, 2)
        if nxt < NCHUNK:
            crn.wait()
            ctn = fire_t(nxt, nb)
        group(ch, b, 3)
        group(ch, b, 4)
        group(ch, b, 5)
        if nxt < NCHUNK:
            ctn.wait()
            csn = fire_s(nxt, nb)
        group(ch, b, 6)
        group(ch, b, 7)
        if nxt < NCHUNK:
            csn.wait()
            con.wait()
        b = nb

    pltpu.sync_copy(res, out_hbm.at[wid])


@jax.jit
def _run(s, o, r, t, e_flat, re_flat, te_flat):
    s4 = s.astype(jnp.int32).reshape(NW, NCHUNK, CHUNK)
    o4 = o.astype(jnp.int32).reshape(NW, NCHUNK, CHUNK)
    r4 = r.astype(jnp.int32).reshape(NW, NCHUNK, CHUNK)
    t4 = t.astype(jnp.int32).reshape(NW, NCHUNK, CHUNK)

    e2 = e_flat.reshape(-1, EMB)
    re2 = re_flat.reshape(-1, EMB)
    te2 = te_flat.reshape(-1, EMB)

    mesh = plsc.VectorSubcoreMesh(core_axis_name="c", subcore_axis_name="s")
    run = pl.kernel(
        _body,
        out_type=jax.ShapeDtypeStruct((NW, PER_W), jnp.float32),
        mesh=mesh,
        compiler_params=pltpu.CompilerParams(
            needs_layout_passes=False, use_tc_tiling_on_sc=False),
        scratch_types=[
            pltpu.VMEM((NCHUNK, CHUNK), jnp.int32),
            pltpu.VMEM((NCHUNK, CHUNK), jnp.int32),
            pltpu.VMEM((NCHUNK, CHUNK), jnp.int32),
            pltpu.VMEM((NCHUNK, CHUNK), jnp.int32),
            pltpu.VMEM((CHUNK, EMB), jnp.float32),
            pltpu.VMEM((CHUNK, EMB), jnp.float32),
            pltpu.VMEM((CHUNK, EMB), jnp.float32),
            pltpu.VMEM((CHUNK, EMB), jnp.float32),
            pltpu.VMEM((PER_W,), jnp.float32),
            pltpu.SemaphoreType.DMA,
            pltpu.SemaphoreType.DMA,
            pltpu.SemaphoreType.DMA,
            pltpu.SemaphoreType.DMA,
        ],
    )
    out = run(s4, o4, r4, t4, e2, re2, te2)
    return out.reshape(BATCH)


_flatten = jax.jit(lambda x: jnp.reshape(x, (-1,)))

_table_cache = [None, None, None]


def _flat_cached(slot, x):
    ent = _table_cache[slot]
    if ent is not None and ent[0] is x:
        return ent[1]
    flat = _flatten(x)
    _table_cache[slot] = (x, flat)
    return flat


def kernel(s, o, r, t, e_embed, r_embed, t_embed):
    e_flat = _flat_cached(0, e_embed)
    re_flat = _flat_cached(1, r_embed)
    te_flat = _flat_cached(2, t_embed)
    return _run(s, o, r, t, e_flat, re_flat, te_flat)


# speedup vs baseline: 1.0015x; 1.0015x over previous
import jax
import jax.numpy as jnp
from jax import lax
from jax.experimental import pallas as pl
from jax.experimental.pallas import tpu as pltpu
from jax.experimental.pallas import tpu_sc as plsc

EMB = 64
BATCH = 16384
NC = 2
NS = 16
NW = NC * NS
PER_W = BATCH // NW
CHUNK = 128
NCHUNK = PER_W // CHUNK
GROUPS = CHUNK // 16


def _body(s_hbm, o_hbm, r_hbm, t_hbm, e_hbm, re_hbm, te_hbm, out_hbm,
          s_idx, o_idx, r_idx, t_idx, acc0, acc1, ob0, ob1,
          res, sem_a0, sem_a1, sem_o0, sem_o1):
    wid = lax.axis_index("s") * NC + lax.axis_index("c")

    pltpu.sync_copy(s_hbm.at[wid], s_idx)
    pltpu.sync_copy(o_hbm.at[wid], o_idx)
    pltpu.sync_copy(r_hbm.at[wid], r_idx)
    pltpu.sync_copy(t_hbm.at[wid], t_idx)

    accs = (acc0, acc1)
    obs = (ob0, ob1)
    sems_a = (sem_a0, sem_a1)
    sems_o = (sem_o0, sem_o1)

    iota = lax.iota(jnp.int32, 16)

    def fire_ro(ch, b):
        cr = pltpu.async_copy(re_hbm.at[r_idx.at[ch]], accs[b], sems_a[b])
        co = pltpu.async_copy(e_hbm.at[o_idx.at[ch]], obs[b], sems_o[b])
        return cr, co

    def fire_t(ch, b):
        return pltpu.async_copy(te_hbm.at[t_idx.at[ch]], accs[b], sems_a[b],
                                add=True)

    def fire_s(ch, b):
        return pltpu.async_copy(e_hbm.at[s_idx.at[ch]], accs[b], sems_a[b],
                                add=True)

    def group(ch, b, g):
        rid = iota + (g * 16)

        def col_body(j, carry):
            acc, col = carry
            va = plsc.load_gather(accs[b], [rid, col])
            vo = plsc.load_gather(obs[b], [rid, col])
            return acc + jnp.abs(va - vo), (col + 1) & 63

        (acc, _) = plsc.parallel_loop(
            0, EMB, carry=(jnp.zeros((16,), jnp.float32), iota),
            unroll=8)(col_body)
        res[pl.ds(ch * CHUNK + g * 16, 16)] = acc

    cr, co = fire_ro(0, 0)
    cr.wait()
    fire_t(0, 0).wait()
    fire_s(0, 0).wait()
    co.wait()

    b = 0
    for ch in range(NCHUNK):
        nxt = ch + 1
        nb = 1 - b
        if nxt < NCHUNK:
            crn, con = fire_ro(nxt, nb)
        group(ch, b, 0)
        group(ch, b, 1)
        group(ch, b, 2)
        if nxt < NCHUNK:
            crn.wait()
            ctn = fire_t(nxt, nb)
        group(ch, b, 3)
        group(ch, b, 4)
        group(ch, b, 5)
        if nxt < NCHUNK:
            ctn.wait()
            csn = fire_s(nxt, nb)
        group(ch, b, 6)
        group(ch, b, 7)
        if nxt < NCHUNK:
            csn.wait()
            con.wait()
        b = nb

    pltpu.sync_copy(res, out_hbm.at[wid])


@jax.jit
def _run(s, o, r, t, e2, re2, te2):
    s4 = s.astype(jnp.int32).reshape(NW, NCHUNK, CHUNK)
    o4 = o.astype(jnp.int32).reshape(NW, NCHUNK, CHUNK)
    r4 = r.astype(jnp.int32).reshape(NW, NCHUNK, CHUNK)
    t4 = t.astype(jnp.int32).reshape(NW, NCHUNK, CHUNK)

    mesh = plsc.VectorSubcoreMesh(core_axis_name="c", subcore_axis_name="s")
    run = pl.kernel(
        _body,
        out_type=jax.ShapeDtypeStruct((NW, PER_W), jnp.float32),
        mesh=mesh,
        compiler_params=pltpu.CompilerParams(
            needs_layout_passes=False, use_tc_tiling_on_sc=False),
        scratch_types=[
            pltpu.VMEM((NCHUNK, CHUNK), jnp.int32),
            pltpu.VMEM((NCHUNK, CHUNK), jnp.int32),
            pltpu.VMEM((NCHUNK, CHUNK), jnp.int32),
            pltpu.VMEM((NCHUNK, CHUNK), jnp.int32),
            pltpu.VMEM((CHUNK, EMB), jnp.float32),
            pltpu.VMEM((CHUNK, EMB), jnp.float32),
            pltpu.VMEM((CHUNK, EMB), jnp.float32),
            pltpu.VMEM((CHUNK, EMB), jnp.float32),
            pltpu.VMEM((PER_W,), jnp.float32),
            pltpu.SemaphoreType.DMA,
            pltpu.SemaphoreType.DMA,
            pltpu.SemaphoreType.DMA,
            pltpu.SemaphoreType.DMA,
        ],
    )
    out = run(s4, o4, r4, t4, e2, re2, te2)
    return out.reshape(BATCH)


def kernel(s, o, r, t, e_embed, r_embed, t_embed):
    return _run(s, o, r, t, e_embed, r_embed, t_embed)
